# baseline (speedup 1.0000x reference)
import functools

import jax
import jax.numpy as jnp
from jax import lax
from jax.experimental import pallas as pl
from jax.experimental.pallas import tpu as pltpu
from jax.experimental.pallas import tpu_sc as plsc

_NC = 2
_NS = 16
_NW = _NC * _NS
_L = 16

_B = 4096
_S = 200
_D = 64
_V = 1000000
_TOTAL = _B * _S
_CHUNK = 128
_CPW = _TOTAL // (_NW * _CHUNK)
_BT = _B // _CHUNK
_EPS = 1e-5

_TBLK = 2048


def _relayout_body(tt_ref, out_ref):
    blk = tt_ref[...]
    out_ref[:, 0:_D] = jnp.transpose(blk, (1, 0))


@jax.jit
def _relayout(table_t):
    return pl.pallas_call(
        _relayout_body,
        grid=((_V + _TBLK - 1) // _TBLK,),
        in_specs=[pl.BlockSpec((_D, _TBLK), lambda g: (0, g))],
        out_specs=pl.BlockSpec((_TBLK, 2 * _D), lambda g: (g, 0)),
        out_shape=jax.ShapeDtypeStruct((_V, 2 * _D), jnp.float32),
    )(table_t)


def _make_perms():
    lanes = lax.iota(jnp.int32, 16)
    return [lax.reshape(lanes ^ k, (16, 1)) for k in (1, 2, 4, 8)]


_GDN = lax.GatherDimensionNumbers(
    offset_dims=(), collapsed_slice_dims=(0,), start_index_map=(0,))


def _allsum16(v, perms):
    for perm in perms:
        v = v + lax.gather(v, perm, _GDN, slice_sizes=(1,),
                           mode=lax.GatherScatterMode.PROMISE_IN_BOUNDS)
    return v


def _rsqrt16(x):
    i = lax.bitcast_convert_type(x, jnp.int32)
    i = jnp.int32(0x5F3759DF) - lax.shift_right_arithmetic(i, 1)
    y = lax.bitcast_convert_type(i, jnp.float32)
    nhalf_x = x * jnp.float32(-0.5)
    for _ in range(3):
        y = y * (jnp.float32(1.5) + nhalf_x * y * y)
    return y


def _make_sc_kernel():
    mesh = plsc.VectorSubcoreMesh(core_axis_name="c", subcore_axis_name="s")

    @functools.partial(
        pl.kernel,
        out_type=jax.ShapeDtypeStruct((_S, _D // 8, _BT, 8, _CHUNK),
                                      jnp.float32),
        mesh=mesh,
        compiler_params=pltpu.CompilerParams(
            use_tc_tiling_on_sc=False, needs_layout_passes=False),
        scratch_types=[
            pltpu.VMEM((_CPW, _CHUNK), jnp.int32),
            pltpu.VMEM((2, _CHUNK, _D), jnp.float32),
            pltpu.VMEM((2, _D, 129), jnp.float32),
            pltpu.VMEM((_D,), jnp.float32),
            pltpu.VMEM((_D,), jnp.float32),
            pltpu.SemaphoreType.DMA,
            pltpu.SemaphoreType.DMA,
            pltpu.SemaphoreType.DMA,
            pltpu.SemaphoreType.DMA,
        ],
    )
    def k(ids_hbm, t2m_hbm, gamma_hbm, beta_hbm, out_hbm,
          idx_v, rows_v, tbuf_v, g_v, b_v, sg0, sg1, so0, so1):
        wid = lax.axis_index("s") * _NC + lax.axis_index("c")
        base_chunk = wid * _CPW
        perms = _make_perms()

        pltpu.sync_copy(ids_hbm.at[pl.ds(base_chunk, _CPW)], idx_v)
        pltpu.sync_copy(gamma_hbm, g_v)
        pltpu.sync_copy(beta_hbm, b_v)

        gs = [g_v[16 * t:16 * (t + 1)] for t in range(4)]
        bs = [b_v[16 * t:16 * (t + 1)] for t in range(4)]
        lanes = lax.iota(jnp.int32, 16)
        row_ids = [lanes + 16 * t for t in range(4)]

        gsems = (sg0, sg1)
        osems = (so0, so1)

        def gather(j, slot):
            pltpu.async_copy(
                t2m_hbm.at[idx_v.at[j]], rows_v.at[slot], gsems[slot])

        def drain_gather(slot):
            pltpu.make_async_copy(
                t2m_hbm.at[pl.ds(0, _CHUNK)], rows_v.at[slot],
                gsems[slot]).wait()

        def compute(slot):
            rows = rows_v.at[slot]
            tbuf = tbuf_v.at[slot]

            def row_body(i, carry):
                r0 = rows[i, 0:16]
                r1 = rows[i, 16:32]
                r2 = rows[i, 32:48]
                r3 = rows[i, 48:64]
                s = (r0 + r1) + (r2 + r3)
                q = (r0 * r0 + r1 * r1) + (r2 * r2 + r3 * r3)
                mv = _allsum16(s, perms) * jnp.float32(1.0 / 64.0)
                qv = _allsum16(q, perms) * jnp.float32(1.0 / 64.0)
                xv = qv - mv * mv + jnp.float32(_EPS)
                yv = _rsqrt16(xv)
                col = jnp.full((_L,), i, dtype=jnp.int32)
                for t, r in enumerate((r0, r1, r2, r3)):
                    val = (r - mv) * yv * gs[t] + bs[t]
                    plsc.store_scatter(tbuf, [row_ids[t], col], val)
                return carry

            lax.fori_loop(0, _CHUNK, row_body, 0, unroll=2)

        def writeout(j, slot):
            c = base_chunk + j
            s_idx = c // _BT
            bt = c % _BT
            for dt in range(8):
                pltpu.async_copy(
                    tbuf_v.at[slot, pl.ds(8 * dt, 8), pl.ds(0, _CHUNK)],
                    out_hbm.at[s_idx, dt, bt], osems[slot])

        def drain_writeout(slot):
            for dt in range(8):
                pltpu.make_async_copy(
                    out_hbm.at[0, dt, 0],
                    tbuf_v.at[slot, pl.ds(8 * dt, 8), pl.ds(0, _CHUNK)],
                    osems[slot]).wait()

        npairs = _CPW // 2
        gather(0, 0)

        def pair_body(p, carry):
            j0 = 2 * p
            drain_gather(0)
            gather(j0 + 1, 1)

            @pl.when(p >= 1)
            def _():
                drain_writeout(0)
            compute(0)
            writeout(j0, 0)
            drain_gather(1)

            @pl.when(p + 1 < npairs)
            def _():
                gather(j0 + 2, 0)

            @pl.when(p >= 1)
            def _():
                drain_writeout(1)
            compute(1)
            writeout(j0 + 1, 1)
            return carry

        lax.fori_loop(0, npairs, pair_body, 0)
        drain_writeout(0)
        drain_writeout(1)

    return k


_sc_kernel = _make_sc_kernel()


def kernel(input_ids, table, gamma, beta):
    tpad = _relayout(table.T)
    t2m = tpad.reshape(2 * _V, _D)
    ids2x = (input_ids * 2).T.reshape(_S * _BT, _CHUNK)
    out5 = _sc_kernel(ids2x, t2m, gamma, beta)
    return jnp.transpose(out5, (2, 4, 0, 1, 3)).reshape(_B, _S, _D)


# speedup vs baseline: 1.9074x; 1.9074x over previous
import functools

import jax
import jax.numpy as jnp
from jax import lax
from jax.experimental import pallas as pl
from jax.experimental.pallas import tpu as pltpu
from jax.experimental.pallas import tpu_sc as plsc

_NC = 2
_NS = 16
_NW = _NC * _NS
_L = 16

_B = 4096
_S = 200
_D = 64
_V = 1000000
_TOTAL = _B * _S
_CHUNK = 128
_CPW = _TOTAL // (_NW * _CHUNK)
_BT = _B // _CHUNK
_EPS = 1e-5

_TBLK = 2048


def _ln_relayout_body(tt_ref, g_ref, b_ref, out_ref):
    blk = tt_ref[...]
    s = jnp.sum(blk, axis=0)
    q = jnp.sum(blk * blk, axis=0)
    m = s * jnp.float32(1.0 / _D)
    var = q * jnp.float32(1.0 / _D) - m * m
    r = lax.rsqrt(var + jnp.float32(_EPS))
    z = (blk - m[None, :]) * r[None, :] * g_ref[...] + b_ref[...]
    out_ref[:, 0:_D] = jnp.transpose(z, (1, 0))


@jax.jit
def _ln_relayout(table_t, gamma2d, beta2d):
    return pl.pallas_call(
        _ln_relayout_body,
        grid=((_V + _TBLK - 1) // _TBLK,),
        in_specs=[
            pl.BlockSpec((_D, _TBLK), lambda g: (0, g)),
            pl.BlockSpec((_D, 1), lambda g: (0, 0)),
            pl.BlockSpec((_D, 1), lambda g: (0, 0)),
        ],
        out_specs=pl.BlockSpec((_TBLK, 2 * _D), lambda g: (g, 0)),
        out_shape=jax.ShapeDtypeStruct((_V, 2 * _D), jnp.float32),
    )(table_t, gamma2d, beta2d)


def _make_sc_kernel():
    mesh = plsc.VectorSubcoreMesh(core_axis_name="c", subcore_axis_name="s")

    @functools.partial(
        pl.kernel,
        out_type=jax.ShapeDtypeStruct((_S, _D // 8, _BT, 8, _CHUNK),
                                      jnp.float32),
        mesh=mesh,
        compiler_params=pltpu.CompilerParams(
            use_tc_tiling_on_sc=False, needs_layout_passes=False),
        scratch_types=[
            pltpu.VMEM((_CPW, _CHUNK), jnp.int32),
            pltpu.VMEM((2, _CHUNK, _D), jnp.float32),
            pltpu.VMEM((2, _D, 129), jnp.float32),
            pltpu.SemaphoreType.DMA,
            pltpu.SemaphoreType.DMA,
            pltpu.SemaphoreType.DMA,
            pltpu.SemaphoreType.DMA,
        ],
    )
    def k(ids_hbm, t2m_hbm, out_hbm,
          idx_v, rows_v, tbuf_v, sg0, sg1, so0, so1):
        wid = lax.axis_index("s") * _NC + lax.axis_index("c")
        base_chunk = wid * _CPW

        pltpu.sync_copy(ids_hbm.at[pl.ds(base_chunk, _CPW)], idx_v)

        lanes = lax.iota(jnp.int32, 16)
        row_ids = [lanes + 16 * t for t in range(4)]

        gsems = (sg0, sg1)
        osems = (so0, so1)

        def gather(j, slot):
            pltpu.async_copy(
                t2m_hbm.at[idx_v.at[j]], rows_v.at[slot], gsems[slot])

        def drain_gather(slot):
            pltpu.make_async_copy(
                t2m_hbm.at[pl.ds(0, _CHUNK)], rows_v.at[slot],
                gsems[slot]).wait()

        def compute(slot):
            rows = rows_v.at[slot]
            tbuf = tbuf_v.at[slot]

            def row_body(i, carry):
                col = jnp.full((_L,), i, dtype=jnp.int32)
                for t in range(4):
                    val = rows[i, 16 * t:16 * (t + 1)]
                    plsc.store_scatter(tbuf, [row_ids[t], col], val)
                return carry

            lax.fori_loop(0, _CHUNK, row_body, 0, unroll=4)

        def writeout(j, slot):
            c = base_chunk + j
            s_idx = c // _BT
            bt = c % _BT
            for dt in range(8):
                pltpu.async_copy(
                    tbuf_v.at[slot, pl.ds(8 * dt, 8), pl.ds(0, _CHUNK)],
                    out_hbm.at[s_idx, dt, bt], osems[slot])

        def drain_writeout(slot):
            for dt in range(8):
                pltpu.make_async_copy(
                    out_hbm.at[0, dt, 0],
                    tbuf_v.at[slot, pl.ds(8 * dt, 8), pl.ds(0, _CHUNK)],
                    osems[slot]).wait()

        npairs = _CPW // 2
        gather(0, 0)

        def pair_body(p, carry):
            j0 = 2 * p
            drain_gather(0)
            gather(j0 + 1, 1)

            @pl.when(p >= 1)
            def _():
                drain_writeout(0)
            compute(0)
            writeout(j0, 0)
            drain_gather(1)

            @pl.when(p + 1 < npairs)
            def _():
                gather(j0 + 2, 0)

            @pl.when(p >= 1)
            def _():
                drain_writeout(1)
            compute(1)
            writeout(j0 + 1, 1)
            return carry

        lax.fori_loop(0, npairs, pair_body, 0)
        drain_writeout(0)
        drain_writeout(1)

    return k


_sc_kernel = _make_sc_kernel()


def kernel(input_ids, table, gamma, beta):
    zpad = _ln_relayout(table.T, gamma.reshape(_D, 1),
                        beta.reshape(_D, 1))
    t2m = zpad.reshape(2 * _V, _D)
    ids2x = (input_ids * 2).T.reshape(_S * _BT, _CHUNK)
    out5 = _sc_kernel(ids2x, t2m)
    return jnp.transpose(out5, (2, 4, 0, 1, 3)).reshape(_B, _S, _D)


# speedup vs baseline: 2.2729x; 1.1916x over previous
import functools

import jax
import jax.numpy as jnp
from jax import lax
from jax.experimental import pallas as pl
from jax.experimental.pallas import tpu as pltpu
from jax.experimental.pallas import tpu_sc as plsc

_NC = 2
_NS = 16
_NW = _NC * _NS
_L = 16

_B = 4096
_S = 200
_D = 64
_V = 1000000
_TOTAL = _B * _S
_CHUNK = 128
_CPW = _TOTAL // (_NW * _CHUNK)
_BT = _B // _CHUNK
_EPS = 1e-5

_TBLK = 2048


_GRID1 = (_V + 2 * _TBLK - 1) // (2 * _TBLK)
_VP = 2 * _TBLK * _GRID1
_NBLK = (_V + _TBLK - 1) // _TBLK


def _ln_relayout_body(t1_ref, t2_ref, g_ref, b_ref, out_ref):
    for half, ref in enumerate((t1_ref, t2_ref)):
        blk = ref[...]
        s = jnp.sum(blk, axis=0)
        q = jnp.sum(blk * blk, axis=0)
        m = s * jnp.float32(1.0 / _D)
        var = q * jnp.float32(1.0 / _D) - m * m
        r = lax.rsqrt(var + jnp.float32(_EPS))
        z = (blk - m[None, :]) * r[None, :] * g_ref[...] + b_ref[...]
        out_ref[:, _D * half:_D * (half + 1)] = jnp.transpose(z, (1, 0))


@jax.jit
def _ln_relayout(table_t, gamma2d, beta2d):
    return pl.pallas_call(
        _ln_relayout_body,
        grid=(_GRID1,),
        in_specs=[
            pl.BlockSpec((_D, _TBLK), lambda g: (0, 2 * g)),
            pl.BlockSpec((_D, _TBLK),
                         lambda g: (0, jnp.minimum(2 * g + 1, _NBLK - 1))),
            pl.BlockSpec((_D, 1), lambda g: (0, 0)),
            pl.BlockSpec((_D, 1), lambda g: (0, 0)),
        ],
        out_specs=pl.BlockSpec((_TBLK, 2 * _D), lambda g: (g, 0)),
        out_shape=jax.ShapeDtypeStruct((_VP // 2, 2 * _D), jnp.float32),
    )(table_t, table_t, gamma2d, beta2d)


def _make_sc_kernel():
    mesh = plsc.VectorSubcoreMesh(core_axis_name="c", subcore_axis_name="s")

    @functools.partial(
        pl.kernel,
        out_type=jax.ShapeDtypeStruct((_S, _D // 8, _BT, 8, _CHUNK),
                                      jnp.float32),
        mesh=mesh,
        compiler_params=pltpu.CompilerParams(
            use_tc_tiling_on_sc=False, needs_layout_passes=False),
        scratch_types=[
            pltpu.VMEM((_CPW, _CHUNK), jnp.int32),
            pltpu.VMEM((2, _CHUNK, _D), jnp.float32),
            pltpu.VMEM((2, _D, 129), jnp.float32),
            pltpu.SemaphoreType.DMA,
            pltpu.SemaphoreType.DMA,
            pltpu.SemaphoreType.DMA,
            pltpu.SemaphoreType.DMA,
        ],
    )
    def k(ids_hbm, t2m_hbm, out_hbm,
          idx_v, rows_v, tbuf_v, sg0, sg1, so0, so1):
        wid = lax.axis_index("s") * _NC + lax.axis_index("c")
        base_chunk = wid * _CPW

        pltpu.sync_copy(ids_hbm.at[pl.ds(base_chunk, _CPW)], idx_v)

        lanes = lax.iota(jnp.int32, 16)
        row_ids = [lanes + 16 * t for t in range(4)]

        gsems = (sg0, sg1)
        osems = (so0, so1)

        def gather(j, slot):
            pltpu.async_copy(
                t2m_hbm.at[idx_v.at[j]], rows_v.at[slot], gsems[slot])

        def drain_gather(slot):
            pltpu.make_async_copy(
                t2m_hbm.at[pl.ds(0, _CHUNK)], rows_v.at[slot],
                gsems[slot]).wait()

        def compute(slot):
            rows = rows_v.at[slot]
            tbuf = tbuf_v.at[slot]

            def row_body(i, carry):
                col = jnp.full((_L,), i, dtype=jnp.int32)
                for t in range(4):
                    val = rows[i, 16 * t:16 * (t + 1)]
                    plsc.store_scatter(tbuf, [row_ids[t], col], val)
                return carry

            lax.fori_loop(0, _CHUNK, row_body, 0, unroll=4)

        def writeout(j, slot):
            c = base_chunk + j
            s_idx = c // _BT
            bt = c % _BT
            for dt in range(8):
                pltpu.async_copy(
                    tbuf_v.at[slot, pl.ds(8 * dt, 8), pl.ds(0, _CHUNK)],
                    out_hbm.at[s_idx, dt, bt], osems[slot])

        def drain_writeout(slot):
            for dt in range(8):
                pltpu.make_async_copy(
                    out_hbm.at[0, dt, 0],
                    tbuf_v.at[slot, pl.ds(8 * dt, 8), pl.ds(0, _CHUNK)],
                    osems[slot]).wait()

        npairs = _CPW // 2
        gather(0, 0)

        def pair_body(p, carry):
            j0 = 2 * p
            drain_gather(0)
            gather(j0 + 1, 1)

            @pl.when(p >= 1)
            def _():
                drain_writeout(0)
            compute(0)
            writeout(j0, 0)
            drain_gather(1)

            @pl.when(p + 1 < npairs)
            def _():
                gather(j0 + 2, 0)

            @pl.when(p >= 1)
            def _():
                drain_writeout(1)
            compute(1)
            writeout(j0 + 1, 1)
            return carry

        lax.fori_loop(0, npairs, pair_body, 0)
        drain_writeout(0)
        drain_writeout(1)

    return k


_sc_kernel = _make_sc_kernel()


def kernel(input_ids, table, gamma, beta):
    zpad = _ln_relayout(table.T, gamma.reshape(_D, 1),
                        beta.reshape(_D, 1))
    t2m = zpad.reshape(_VP, _D)
    q = input_ids >> 11
    i = input_ids & 2047
    idx2 = ((q >> 1) << 12) + (i << 1) + (q & 1)
    ids2x = idx2.T.reshape(_S * _BT, _CHUNK)
    out5 = _sc_kernel(ids2x, t2m)
    return jnp.transpose(out5, (2, 4, 0, 1, 3)).reshape(_B, _S, _D)


# speedup vs baseline: 2.4566x; 1.0808x over previous
import functools

import jax
import jax.numpy as jnp
from jax import lax
from jax.experimental import pallas as pl
from jax.experimental.pallas import tpu as pltpu
from jax.experimental.pallas import tpu_sc as plsc

_NC = 2
_NS = 16
_NW = _NC * _NS
_L = 16

_B = 4096
_S = 200
_D = 64
_V = 1000000
_TOTAL = _B * _S
_CHUNK = 256
_CPW = _TOTAL // (_NW * _CHUNK)
_BT = _B // 128
_CT = _B // _CHUNK
_HB = _CHUNK // 128
_EPS = 1e-5

_TBLK = 4096
_LB = _TBLK.bit_length() - 1


_GRID1 = (_V + 2 * _TBLK - 1) // (2 * _TBLK)
_VP = 2 * _TBLK * _GRID1
_NBLK = (_V + _TBLK - 1) // _TBLK


def _ln_relayout_body(t1_ref, t2_ref, g_ref, b_ref, out_ref):
    for half, ref in enumerate((t1_ref, t2_ref)):
        blk = ref[...]
        s = jnp.sum(blk, axis=0)
        q = jnp.sum(blk * blk, axis=0)
        m = s * jnp.float32(1.0 / _D)
        var = q * jnp.float32(1.0 / _D) - m * m
        r = lax.rsqrt(var + jnp.float32(_EPS))
        z = (blk - m[None, :]) * r[None, :] * g_ref[...] + b_ref[...]
        out_ref[:, _D * half:_D * (half + 1)] = jnp.transpose(z, (1, 0))


@jax.jit
def _ln_relayout(table_t, gamma2d, beta2d):
    return pl.pallas_call(
        _ln_relayout_body,
        grid=(_GRID1,),
        in_specs=[
            pl.BlockSpec((_D, _TBLK), lambda g: (0, 2 * g)),
            pl.BlockSpec((_D, _TBLK),
                         lambda g: (0, jnp.minimum(2 * g + 1, _NBLK - 1))),
            pl.BlockSpec((_D, 1), lambda g: (0, 0)),
            pl.BlockSpec((_D, 1), lambda g: (0, 0)),
        ],
        out_specs=pl.BlockSpec((_TBLK, 2 * _D), lambda g: (g, 0)),
        out_shape=jax.ShapeDtypeStruct((_VP // 2, 2 * _D), jnp.float32),
    )(table_t, table_t, gamma2d, beta2d)


def _make_sc_kernel():
    mesh = plsc.VectorSubcoreMesh(core_axis_name="c", subcore_axis_name="s")

    @functools.partial(
        pl.kernel,
        out_type=jax.ShapeDtypeStruct((_S, _D // 8, _BT, 8, 128),
                                      jnp.float32),
        mesh=mesh,
        compiler_params=pltpu.CompilerParams(
            use_tc_tiling_on_sc=False, needs_layout_passes=False),
        scratch_types=[
            pltpu.VMEM((_CPW, _CHUNK), jnp.int32),
            pltpu.VMEM((2, _CHUNK, _D), jnp.float32),
            pltpu.VMEM((2, _D, _CHUNK + 1), jnp.float32),
            pltpu.SemaphoreType.DMA,
            pltpu.SemaphoreType.DMA,
            pltpu.SemaphoreType.DMA,
            pltpu.SemaphoreType.DMA,
        ],
    )
    def k(ids_hbm, t2m_hbm, out_hbm,
          idx_v, rows_v, tbuf_v, sg0, sg1, so0, so1):
        wid = lax.axis_index("s") * _NC + lax.axis_index("c")
        base_chunk = wid * _CPW

        pltpu.sync_copy(ids_hbm.at[pl.ds(base_chunk, _CPW)], idx_v)

        lanes = lax.iota(jnp.int32, 16)
        row_ids = [lanes + 16 * t for t in range(4)]

        gsems = (sg0, sg1)
        osems = (so0, so1)

        def gather(j, slot):
            pltpu.async_copy(
                t2m_hbm.at[idx_v.at[j]], rows_v.at[slot], gsems[slot])

        def drain_gather(slot):
            pltpu.make_async_copy(
                t2m_hbm.at[pl.ds(0, _CHUNK)], rows_v.at[slot],
                gsems[slot]).wait()

        def compute(slot):
            rows = rows_v.at[slot]
            tbuf = tbuf_v.at[slot]

            def row_body(i, carry):
                col = jnp.full((_L,), i, dtype=jnp.int32)
                for t in range(4):
                    val = rows[i, 16 * t:16 * (t + 1)]
                    plsc.store_scatter(tbuf, [row_ids[t], col], val)
                return carry

            lax.fori_loop(0, _CHUNK, row_body, 0, unroll=4)

        def writeout(j, slot):
            c = base_chunk + j
            s_idx = c // _CT
            bt2 = c % _CT
            for dt in range(8):
                for h in range(_HB):
                    pltpu.async_copy(
                        tbuf_v.at[slot, pl.ds(8 * dt, 8), pl.ds(128 * h, 128)],
                        out_hbm.at[s_idx, dt, bt2 * _HB + h], osems[slot])

        def drain_writeout(slot):
            for dt in range(8):
                for h in range(_HB):
                    pltpu.make_async_copy(
                        out_hbm.at[0, dt, 0],
                        tbuf_v.at[slot, pl.ds(8 * dt, 8), pl.ds(128 * h, 128)],
                        osems[slot]).wait()

        npairs = _CPW // 2
        gather(0, 0)

        def pair_body(p, carry):
            j0 = 2 * p
            drain_gather(0)
            gather(j0 + 1, 1)

            @pl.when(p >= 1)
            def _():
                drain_writeout(0)
            compute(0)
            writeout(j0, 0)
            drain_gather(1)

            @pl.when(p + 1 < npairs)
            def _():
                gather(j0 + 2, 0)

            @pl.when(p >= 1)
            def _():
                drain_writeout(1)
            compute(1)
            writeout(j0 + 1, 1)
            return carry

        lax.fori_loop(0, npairs, pair_body, 0)
        drain_writeout(0)
        drain_writeout(1)

    return k


_sc_kernel = _make_sc_kernel()


def kernel(input_ids, table, gamma, beta):
    zpad = _ln_relayout(table.T, gamma.reshape(_D, 1),
                        beta.reshape(_D, 1))
    t2m = zpad.reshape(_VP, _D)
    q = input_ids >> _LB
    i = input_ids & (_TBLK - 1)
    idx2 = ((q >> 1) << (_LB + 1)) + (i << 1) + (q & 1)
    ids2x = idx2.T.reshape(_S * _CT, _CHUNK)
    out5 = _sc_kernel(ids2x, t2m)
    return jnp.transpose(out5, (2, 4, 0, 1, 3)).reshape(_B, _S, _D)


# speedup vs baseline: 2.5203x; 1.0259x over previous
import functools

import jax
import jax.numpy as jnp
from jax import lax
from jax.experimental import pallas as pl
from jax.experimental.pallas import tpu as pltpu
from jax.experimental.pallas import tpu_sc as plsc

_NC = 2
_NS = 16
_NW = _NC * _NS
_L = 16

_B = 4096
_S = 200
_D = 64
_V = 1000000
_TOTAL = _B * _S
_CHUNK = 128
_CPW = _TOTAL // (_NW * _CHUNK)
_BT = _B // 128
_CT = _B // _CHUNK
_HB = _CHUNK // 128
_EPS = 1e-5

_TBLK = 4096
_LB = _TBLK.bit_length() - 1


_GRID1 = (_V + 2 * _TBLK - 1) // (2 * _TBLK)
_VP = 2 * _TBLK * _GRID1
_NBLK = (_V + _TBLK - 1) // _TBLK


def _ln_relayout_body(t1_ref, t2_ref, g_ref, b_ref, out_ref):
    for half, ref in enumerate((t1_ref, t2_ref)):
        blk = ref[...]
        s = jnp.sum(blk, axis=0)
        q = jnp.sum(blk * blk, axis=0)
        m = s * jnp.float32(1.0 / _D)
        var = q * jnp.float32(1.0 / _D) - m * m
        r = lax.rsqrt(var + jnp.float32(_EPS))
        z = (blk - m[None, :]) * r[None, :] * g_ref[...] + b_ref[...]
        out_ref[:, _D * half:_D * (half + 1)] = jnp.transpose(z, (1, 0))


@jax.jit
def _ln_relayout(table_t, gamma2d, beta2d):
    return pl.pallas_call(
        _ln_relayout_body,
        grid=(_GRID1,),
        in_specs=[
            pl.BlockSpec((_D, _TBLK), lambda g: (0, 2 * g)),
            pl.BlockSpec((_D, _TBLK),
                         lambda g: (0, jnp.minimum(2 * g + 1, _NBLK - 1))),
            pl.BlockSpec((_D, 1), lambda g: (0, 0)),
            pl.BlockSpec((_D, 1), lambda g: (0, 0)),
        ],
        out_specs=pl.BlockSpec((_TBLK, 2 * _D), lambda g: (g, 0)),
        out_shape=jax.ShapeDtypeStruct((_VP // 2, 2 * _D), jnp.float32),
    )(table_t, table_t, gamma2d, beta2d)


def _make_sc_kernel():
    mesh = plsc.VectorSubcoreMesh(core_axis_name="c", subcore_axis_name="s")

    @functools.partial(
        pl.kernel,
        out_type=jax.ShapeDtypeStruct((_S, _D // 8, _BT, 8, 128),
                                      jnp.float32),
        mesh=mesh,
        compiler_params=pltpu.CompilerParams(
            use_tc_tiling_on_sc=False, needs_layout_passes=False),
        scratch_types=[
            pltpu.VMEM((_CPW, _CHUNK), jnp.int32),
            pltpu.VMEM((2, _CHUNK, _D), jnp.float32),
            pltpu.VMEM((2, _D, _CHUNK + 1), jnp.float32),
            pltpu.SemaphoreType.DMA,
            pltpu.SemaphoreType.DMA,
            pltpu.SemaphoreType.DMA,
            pltpu.SemaphoreType.DMA,
        ],
    )
    def k(ids_hbm, t2m_hbm, out_hbm,
          idx_v, rows_v, tbuf_v, sg0, sg1, so0, so1):
        wid = lax.axis_index("s") * _NC + lax.axis_index("c")
        base_chunk = wid * _CPW

        pltpu.sync_copy(ids_hbm.at[pl.ds(base_chunk, _CPW)], idx_v)

        lanes = lax.iota(jnp.int32, 16)
        row_ids = [lanes + 16 * t for t in range(4)]

        gsems = (sg0, sg1)
        osems = (so0, so1)

        def gather(j, slot):
            pltpu.async_copy(
                t2m_hbm.at[idx_v.at[j]], rows_v.at[slot], gsems[slot])

        def drain_gather(slot):
            pltpu.make_async_copy(
                t2m_hbm.at[pl.ds(0, _CHUNK)], rows_v.at[slot],
                gsems[slot]).wait()

        def compute(slot):
            rows = rows_v.at[slot]
            tbuf = tbuf_v.at[slot]

            def row_body(i, carry):
                col = jnp.full((_L,), i, dtype=jnp.int32)
                for t in range(4):
                    val = rows[i, 16 * t:16 * (t + 1)]
                    plsc.store_scatter(tbuf, [row_ids[t], col], val)
                return carry

            lax.fori_loop(0, _CHUNK, row_body, 0, unroll=4)

        def writeout(j, slot):
            c = base_chunk + j
            s_idx = c // _CT
            bt2 = c % _CT
            for dt in range(8):
                for h in range(_HB):
                    pltpu.async_copy(
                        tbuf_v.at[slot, pl.ds(8 * dt, 8), pl.ds(128 * h, 128)],
                        out_hbm.at[s_idx, dt, bt2 * _HB + h], osems[slot])

        def drain_writeout(slot):
            for dt in range(8):
                for h in range(_HB):
                    pltpu.make_async_copy(
                        out_hbm.at[0, dt, 0],
                        tbuf_v.at[slot, pl.ds(8 * dt, 8), pl.ds(128 * h, 128)],
                        osems[slot]).wait()

        npairs = _CPW // 2
        gather(0, 0)

        def pair_body(p, carry):
            j0 = 2 * p
            drain_gather(0)
            gather(j0 + 1, 1)

            @pl.when(p >= 1)
            def _():
                drain_writeout(0)
            compute(0)
            writeout(j0, 0)
            drain_gather(1)

            @pl.when(p + 1 < npairs)
            def _():
                gather(j0 + 2, 0)

            @pl.when(p >= 1)
            def _():
                drain_writeout(1)
            compute(1)
            writeout(j0 + 1, 1)
            return carry

        lax.fori_loop(0, npairs, pair_body, 0)
        drain_writeout(0)
        drain_writeout(1)

    return k


_sc_kernel = _make_sc_kernel()


def kernel(input_ids, table, gamma, beta):
    zpad = _ln_relayout(table.T, gamma.reshape(_D, 1),
                        beta.reshape(_D, 1))
    t2m = zpad.reshape(_VP, _D)
    q = input_ids >> _LB
    i = input_ids & (_TBLK - 1)
    idx2 = ((q >> 1) << (_LB + 1)) + (i << 1) + (q & 1)
    ids2x = idx2.T.reshape(_S * _CT, _CHUNK)
    out5 = _sc_kernel(ids2x, t2m)
    return jnp.transpose(out5, (2, 4, 0, 1, 3)).reshape(_B, _S, _D)
